# baseline (speedup 1.0000x reference)
import functools

import jax
import jax.numpy as jnp
from jax import lax
from jax.experimental import pallas as pl
from jax.experimental.pallas import tpu as pltpu
from jax.experimental.pallas import tpu_sc as plsc

KE_DIM = 64
MARGIN_V = 1.0
LANES = 16
NWORK = 32
STAGE = 128
GR = 64
POS_N = 16384
PER_W = POS_N // NWORK
NSTAGE = PER_W // STAGE


def _make_sc_call():
    mesh = plsc.VectorSubcoreMesh(core_axis_name="c", subcore_axis_name="s")

    row_t = pltpu.VMEM((GR, 2 * KE_DIM), jnp.float32)
    idx_t = pltpu.VMEM((STAGE,), jnp.int32)

    @functools.partial(
        pl.kernel,
        mesh=mesh,
        out_type=jax.ShapeDtypeStruct((NWORK, LANES), jnp.float32),
        scratch_types=[
            idx_t, idx_t, idx_t,
            idx_t, idx_t, idx_t,
            idx_t, idx_t, idx_t,
            idx_t, idx_t, idx_t,
            row_t, row_t, row_t, row_t, row_t, row_t,
            row_t, row_t, row_t, row_t, row_t, row_t,
            pltpu.VMEM((LANES,), jnp.float32),
            pltpu.SemaphoreType.DMA,
        ],
    )
    def sc_fn(eEmb, eCov, rEmb, rCov, hIdx, rIdx, tIdx, hBit, rBit, tBit, out,
              phix, prix, ptix, nhix, nrix, ntix,
              phbx, prbx, ptbx, nhbx, nrbx, ntbx,
              phm, phv, ptm, ptv, prm, prv,
              nhm, nhv, ntm, ntv, nrm, nrv,
              accv, sem):
        cid = lax.axis_index("c")
        sid = lax.axis_index("s")
        wid = sid * 2 + cid
        base = wid * PER_W

        iota = lax.iota(jnp.int32, LANES)
        one = jnp.float32(1.0)

        def do_stage(c, loss):
            pos_off = base + c * STAGE
            neg_off = pos_off + POS_N
            pltpu.sync_copy(hIdx.at[pl.ds(pos_off, STAGE)], phix)
            pltpu.sync_copy(rIdx.at[pl.ds(pos_off, STAGE)], prix)
            pltpu.sync_copy(tIdx.at[pl.ds(pos_off, STAGE)], ptix)
            pltpu.sync_copy(hIdx.at[pl.ds(neg_off, STAGE)], nhix)
            pltpu.sync_copy(rIdx.at[pl.ds(neg_off, STAGE)], nrix)
            pltpu.sync_copy(tIdx.at[pl.ds(neg_off, STAGE)], ntix)
            pltpu.sync_copy(hBit.at[pl.ds(pos_off, STAGE)], phbx)
            pltpu.sync_copy(rBit.at[pl.ds(pos_off, STAGE)], prbx)
            pltpu.sync_copy(tBit.at[pl.ds(pos_off, STAGE)], ptbx)
            pltpu.sync_copy(hBit.at[pl.ds(neg_off, STAGE)], nhbx)
            pltpu.sync_copy(rBit.at[pl.ds(neg_off, STAGE)], nrbx)
            pltpu.sync_copy(tBit.at[pl.ds(neg_off, STAGE)], ntbx)

            for sub in range(2):
                so = sub * GR
                sl_i = pl.ds(so, GR)
                cps = [
                    pltpu.async_copy(eEmb.at[phix.at[sl_i]], phm, sem),
                    pltpu.async_copy(eCov.at[phix.at[sl_i]], phv, sem),
                    pltpu.async_copy(eEmb.at[ptix.at[sl_i]], ptm, sem),
                    pltpu.async_copy(eCov.at[ptix.at[sl_i]], ptv, sem),
                    pltpu.async_copy(rEmb.at[prix.at[sl_i]], prm, sem),
                    pltpu.async_copy(rCov.at[prix.at[sl_i]], prv, sem),
                    pltpu.async_copy(eEmb.at[nhix.at[sl_i]], nhm, sem),
                    pltpu.async_copy(eCov.at[nhix.at[sl_i]], nhv, sem),
                    pltpu.async_copy(eEmb.at[ntix.at[sl_i]], ntm, sem),
                    pltpu.async_copy(eCov.at[ntix.at[sl_i]], ntv, sem),
                    pltpu.async_copy(rEmb.at[nrix.at[sl_i]], nrm, sem),
                    pltpu.async_copy(rCov.at[nrix.at[sl_i]], nrv, sem),
                ]
                for cp in cps:
                    cp.wait()

                def body(j, carry):
                    r0 = j * LANES
                    b0 = so + r0
                    bph = phbx[pl.ds(b0, LANES)]
                    bpr = prbx[pl.ds(b0, LANES)]
                    bpt = ptbx[pl.ds(b0, LANES)]
                    bnh = nhbx[pl.ds(b0, LANES)]
                    bnr = nrbx[pl.ds(b0, LANES)]
                    bnt = ntbx[pl.ds(b0, LANES)]
                    for k in range(LANES):
                        t = r0 + k
                        oph = bph[k] * jnp.int32(KE_DIM)
                        opr = bpr[k] * jnp.int32(KE_DIM)
                        opt = bpt[k] * jnp.int32(KE_DIM)
                        onh = bnh[k] * jnp.int32(KE_DIM)
                        onr = bnr[k] * jnp.int32(KE_DIM)
                        ont = bnt[k] * jnp.int32(KE_DIM)
                        d = jnp.zeros((LANES,), jnp.float32)
                        for g in range(KE_DIM // LANES):
                            sl = g * LANES
                            phm_v = phm[t, pl.ds(oph + sl, LANES)]
                            phv_v = phv[t, pl.ds(oph + sl, LANES)]
                            ptm_v = ptm[t, pl.ds(opt + sl, LANES)]
                            ptv_v = ptv[t, pl.ds(opt + sl, LANES)]
                            prm_v = prm[t, pl.ds(opr + sl, LANES)]
                            prv_v = prv[t, pl.ds(opr + sl, LANES)]
                            evp = ptv_v + phv_v
                            dp = prm_v - (ptm_v - phm_v)
                            ddp = dp * dp
                            sp = (evp + ddp) * (one / prv_v) + (prv_v + ddp) * (one / evp)

                            nhm_v = nhm[t, pl.ds(onh + sl, LANES)]
                            nhv_v = nhv[t, pl.ds(onh + sl, LANES)]
                            ntm_v = ntm[t, pl.ds(ont + sl, LANES)]
                            ntv_v = ntv[t, pl.ds(ont + sl, LANES)]
                            nrm_v = nrm[t, pl.ds(onr + sl, LANES)]
                            nrv_v = nrv[t, pl.ds(onr + sl, LANES)]
                            evn = ntv_v + nhv_v
                            dn = nrm_v - (ntm_v - nhm_v)
                            ddn = dn * dn
                            sn = (evn + ddn) * (one / nrv_v) + (nrv_v + ddn) * (one / evn)
                            d = d + (sp - sn)
                        p0 = d[0] + d[1]
                        p1 = d[2] + d[3]
                        p2 = d[4] + d[5]
                        p3 = d[6] + d[7]
                        p4 = d[8] + d[9]
                        p5 = d[10] + d[11]
                        p6 = d[12] + d[13]
                        p7 = d[14] + d[15]
                        tot = ((p0 + p1) + (p2 + p3)) + ((p4 + p5) + (p6 + p7))
                        carry = carry + jnp.maximum(
                            tot * jnp.float32(0.25) + jnp.float32(MARGIN_V),
                            jnp.float32(0.0))
                    return carry

                loss = lax.fori_loop(0, GR // LANES, body, loss)
            return loss

        loss = lax.fori_loop(0, NSTAGE, do_stage, jnp.float32(0.0))

        accv[...] = jnp.where(iota == 0, loss, jnp.float32(0.0))
        pltpu.sync_copy(accv, out.at[wid])

    return sc_fn


_SC_FN = _make_sc_call()


@jax.jit
def kernel(posX, negX, entityEmbed, entityCovar, relationEmbed, relationCovar):
    X = jnp.concatenate([posX, negX], axis=0)
    h = X[:, 0]
    r = X[:, 1]
    t = X[:, 2]
    n2 = entityEmbed.shape[0] // 2
    eEmb = entityEmbed.reshape(n2, 2 * KE_DIM)
    eCov = entityCovar.reshape(n2, 2 * KE_DIM)
    rEmb = relationEmbed.reshape(n2, 2 * KE_DIM)
    rCov = relationCovar.reshape(n2, 2 * KE_DIM)
    partials = _SC_FN(eEmb, eCov, rEmb, rCov,
                      h >> 1, r >> 1, t >> 1, h & 1, r & 1, t & 1)
    return jnp.sum(partials) / jnp.float32(posX.shape[0])


# speedup vs baseline: 1.0013x; 1.0013x over previous
import functools

import jax
import jax.numpy as jnp
from jax import lax
from jax.experimental import pallas as pl
from jax.experimental.pallas import tpu as pltpu
from jax.experimental.pallas import tpu_sc as plsc

KE_DIM = 64
MARGIN_V = 1.0
LANES = 16
NWORK = 32
STAGE = 128
GR = 64
POS_N = 16384
PER_W = POS_N // NWORK
NSTAGE = PER_W // STAGE


def _make_sc_call():
    mesh = plsc.VectorSubcoreMesh(core_axis_name="c", subcore_axis_name="s")

    row_t = pltpu.VMEM((GR, 2 * KE_DIM), jnp.float32)
    idx_t = pltpu.VMEM((STAGE,), jnp.int32)

    @functools.partial(
        pl.kernel,
        mesh=mesh,
        compiler_params=pltpu.CompilerParams(use_tc_tiling_on_sc=True),
        out_type=jax.ShapeDtypeStruct((NWORK, LANES), jnp.float32),
        scratch_types=[
            idx_t, idx_t, idx_t,
            idx_t, idx_t, idx_t,
            idx_t, idx_t, idx_t,
            idx_t, idx_t, idx_t,
            row_t, row_t, row_t, row_t, row_t, row_t,
            row_t, row_t, row_t, row_t, row_t, row_t,
            pltpu.VMEM((LANES,), jnp.float32),
            pltpu.SemaphoreType.DMA,
        ],
    )
    def sc_fn(eEmb, eCov, rEmb, rCov, hIdx, rIdx, tIdx, hBit, rBit, tBit, out,
              phix, prix, ptix, nhix, nrix, ntix,
              phbx, prbx, ptbx, nhbx, nrbx, ntbx,
              phm, phv, ptm, ptv, prm, prv,
              nhm, nhv, ntm, ntv, nrm, nrv,
              accv, sem):
        cid = lax.axis_index("c")
        sid = lax.axis_index("s")
        wid = sid * 2 + cid
        base = wid * PER_W

        iota = lax.iota(jnp.int32, LANES)
        one = jnp.float32(1.0)

        def do_stage(c, loss):
            pos_off = base + c * STAGE
            neg_off = pos_off + POS_N
            pltpu.sync_copy(hIdx.at[pl.ds(pos_off, STAGE)], phix)
            pltpu.sync_copy(rIdx.at[pl.ds(pos_off, STAGE)], prix)
            pltpu.sync_copy(tIdx.at[pl.ds(pos_off, STAGE)], ptix)
            pltpu.sync_copy(hIdx.at[pl.ds(neg_off, STAGE)], nhix)
            pltpu.sync_copy(rIdx.at[pl.ds(neg_off, STAGE)], nrix)
            pltpu.sync_copy(tIdx.at[pl.ds(neg_off, STAGE)], ntix)
            pltpu.sync_copy(hBit.at[pl.ds(pos_off, STAGE)], phbx)
            pltpu.sync_copy(rBit.at[pl.ds(pos_off, STAGE)], prbx)
            pltpu.sync_copy(tBit.at[pl.ds(pos_off, STAGE)], ptbx)
            pltpu.sync_copy(hBit.at[pl.ds(neg_off, STAGE)], nhbx)
            pltpu.sync_copy(rBit.at[pl.ds(neg_off, STAGE)], nrbx)
            pltpu.sync_copy(tBit.at[pl.ds(neg_off, STAGE)], ntbx)

            for sub in range(2):
                so = sub * GR
                sl_i = pl.ds(so, GR)
                cps = [
                    pltpu.async_copy(eEmb.at[phix.at[sl_i]], phm, sem),
                    pltpu.async_copy(eCov.at[phix.at[sl_i]], phv, sem),
                    pltpu.async_copy(eEmb.at[ptix.at[sl_i]], ptm, sem),
                    pltpu.async_copy(eCov.at[ptix.at[sl_i]], ptv, sem),
                    pltpu.async_copy(rEmb.at[prix.at[sl_i]], prm, sem),
                    pltpu.async_copy(rCov.at[prix.at[sl_i]], prv, sem),
                    pltpu.async_copy(eEmb.at[nhix.at[sl_i]], nhm, sem),
                    pltpu.async_copy(eCov.at[nhix.at[sl_i]], nhv, sem),
                    pltpu.async_copy(eEmb.at[ntix.at[sl_i]], ntm, sem),
                    pltpu.async_copy(eCov.at[ntix.at[sl_i]], ntv, sem),
                    pltpu.async_copy(rEmb.at[nrix.at[sl_i]], nrm, sem),
                    pltpu.async_copy(rCov.at[nrix.at[sl_i]], nrv, sem),
                ]
                for cp in cps:
                    cp.wait()

                def body(j, carry):
                    r0 = j * LANES
                    b0 = so + r0
                    bph = phbx[pl.ds(b0, LANES)]
                    bpr = prbx[pl.ds(b0, LANES)]
                    bpt = ptbx[pl.ds(b0, LANES)]
                    bnh = nhbx[pl.ds(b0, LANES)]
                    bnr = nrbx[pl.ds(b0, LANES)]
                    bnt = ntbx[pl.ds(b0, LANES)]
                    for k in range(LANES):
                        t = r0 + k
                        oph = bph[k] * jnp.int32(KE_DIM)
                        opr = bpr[k] * jnp.int32(KE_DIM)
                        opt = bpt[k] * jnp.int32(KE_DIM)
                        onh = bnh[k] * jnp.int32(KE_DIM)
                        onr = bnr[k] * jnp.int32(KE_DIM)
                        ont = bnt[k] * jnp.int32(KE_DIM)
                        d = jnp.zeros((LANES,), jnp.float32)
                        for g in range(KE_DIM // LANES):
                            sl = g * LANES
                            phm_v = phm[t, pl.ds(oph + sl, LANES)]
                            phv_v = phv[t, pl.ds(oph + sl, LANES)]
                            ptm_v = ptm[t, pl.ds(opt + sl, LANES)]
                            ptv_v = ptv[t, pl.ds(opt + sl, LANES)]
                            prm_v = prm[t, pl.ds(opr + sl, LANES)]
                            prv_v = prv[t, pl.ds(opr + sl, LANES)]
                            evp = ptv_v + phv_v
                            dp = prm_v - (ptm_v - phm_v)
                            ddp = dp * dp
                            sp = (evp + ddp) * (one / prv_v) + (prv_v + ddp) * (one / evp)

                            nhm_v = nhm[t, pl.ds(onh + sl, LANES)]
                            nhv_v = nhv[t, pl.ds(onh + sl, LANES)]
                            ntm_v = ntm[t, pl.ds(ont + sl, LANES)]
                            ntv_v = ntv[t, pl.ds(ont + sl, LANES)]
                            nrm_v = nrm[t, pl.ds(onr + sl, LANES)]
                            nrv_v = nrv[t, pl.ds(onr + sl, LANES)]
                            evn = ntv_v + nhv_v
                            dn = nrm_v - (ntm_v - nhm_v)
                            ddn = dn * dn
                            sn = (evn + ddn) * (one / nrv_v) + (nrv_v + ddn) * (one / evn)
                            d = d + (sp - sn)
                        p0 = d[0] + d[1]
                        p1 = d[2] + d[3]
                        p2 = d[4] + d[5]
                        p3 = d[6] + d[7]
                        p4 = d[8] + d[9]
                        p5 = d[10] + d[11]
                        p6 = d[12] + d[13]
                        p7 = d[14] + d[15]
                        tot = ((p0 + p1) + (p2 + p3)) + ((p4 + p5) + (p6 + p7))
                        carry = carry + jnp.maximum(
                            tot * jnp.float32(0.25) + jnp.float32(MARGIN_V),
                            jnp.float32(0.0))
                    return carry

                loss = lax.fori_loop(0, GR // LANES, body, loss)
            return loss

        loss = lax.fori_loop(0, NSTAGE, do_stage, jnp.float32(0.0))

        accv[...] = jnp.where(iota == 0, loss, jnp.float32(0.0))
        pltpu.sync_copy(accv, out.at[wid])

    return sc_fn


_SC_FN = _make_sc_call()


@jax.jit
def kernel(posX, negX, entityEmbed, entityCovar, relationEmbed, relationCovar):
    X = jnp.concatenate([posX, negX], axis=0)
    h = X[:, 0]
    r = X[:, 1]
    t = X[:, 2]
    n2 = entityEmbed.shape[0] // 2
    eEmb = entityEmbed.reshape(n2, 2 * KE_DIM)
    eCov = entityCovar.reshape(n2, 2 * KE_DIM)
    rEmb = relationEmbed.reshape(n2, 2 * KE_DIM)
    rCov = relationCovar.reshape(n2, 2 * KE_DIM)
    partials = _SC_FN(eEmb, eCov, rEmb, rCov,
                      h >> 1, r >> 1, t >> 1, h & 1, r & 1, t & 1)
    return jnp.sum(partials) / jnp.float32(posX.shape[0])


# speedup vs baseline: 1.0521x; 1.0507x over previous
import functools

import jax
import jax.numpy as jnp
from jax import lax
from jax.experimental import pallas as pl
from jax.experimental.pallas import tpu as pltpu
from jax.experimental.pallas import tpu_sc as plsc

KE_DIM = 64
MARGIN_V = 1.0
LANES = 16
NWORK = 32
STAGE = 128
GR = 64
POS_N = 16384
PER_W = POS_N // NWORK
NSTAGE = PER_W // STAGE


def _make_sc_call():
    mesh = plsc.VectorSubcoreMesh(core_axis_name="c", subcore_axis_name="s")

    row_t = pltpu.VMEM((GR, KE_DIM), jnp.float32)
    idx_t = pltpu.VMEM((STAGE,), jnp.int32)

    @functools.partial(
        pl.kernel,
        mesh=mesh,
        compiler_params=pltpu.CompilerParams(use_tc_tiling_on_sc=False),
        out_type=jax.ShapeDtypeStruct((NWORK, LANES), jnp.float32),
        scratch_types=[
            idx_t, idx_t, idx_t,
            idx_t, idx_t, idx_t,
            row_t, row_t, row_t, row_t, row_t, row_t,
            row_t, row_t, row_t, row_t, row_t, row_t,
            pltpu.VMEM((LANES,), jnp.float32),
            pltpu.SemaphoreType.DMA,
        ],
    )
    def sc_fn(eEmb, eCov, rEmb, rCov, hIdx, rIdx, tIdx, out,
              phix, prix, ptix, nhix, nrix, ntix,
              phm, phv, ptm, ptv, prm, prv,
              nhm, nhv, ntm, ntv, nrm, nrv,
              accv, sem):
        cid = lax.axis_index("c")
        sid = lax.axis_index("s")
        wid = sid * 2 + cid
        base = wid * PER_W

        iota = lax.iota(jnp.int32, LANES)
        one = jnp.float32(1.0)

        def do_stage(c, loss):
            pos_off = base + c * STAGE
            neg_off = pos_off + POS_N
            pltpu.sync_copy(hIdx.at[pl.ds(pos_off, STAGE)], phix)
            pltpu.sync_copy(rIdx.at[pl.ds(pos_off, STAGE)], prix)
            pltpu.sync_copy(tIdx.at[pl.ds(pos_off, STAGE)], ptix)
            pltpu.sync_copy(hIdx.at[pl.ds(neg_off, STAGE)], nhix)
            pltpu.sync_copy(rIdx.at[pl.ds(neg_off, STAGE)], nrix)
            pltpu.sync_copy(tIdx.at[pl.ds(neg_off, STAGE)], ntix)

            for sub in range(2):
                so = sub * GR
                sl_i = pl.ds(so, GR)
                cps = [
                    pltpu.async_copy(eEmb.at[phix.at[sl_i]], phm, sem),
                    pltpu.async_copy(eCov.at[phix.at[sl_i]], phv, sem),
                    pltpu.async_copy(eEmb.at[ptix.at[sl_i]], ptm, sem),
                    pltpu.async_copy(eCov.at[ptix.at[sl_i]], ptv, sem),
                    pltpu.async_copy(rEmb.at[prix.at[sl_i]], prm, sem),
                    pltpu.async_copy(rCov.at[prix.at[sl_i]], prv, sem),
                    pltpu.async_copy(eEmb.at[nhix.at[sl_i]], nhm, sem),
                    pltpu.async_copy(eCov.at[nhix.at[sl_i]], nhv, sem),
                    pltpu.async_copy(eEmb.at[ntix.at[sl_i]], ntm, sem),
                    pltpu.async_copy(eCov.at[ntix.at[sl_i]], ntv, sem),
                    pltpu.async_copy(rEmb.at[nrix.at[sl_i]], nrm, sem),
                    pltpu.async_copy(rCov.at[nrix.at[sl_i]], nrv, sem),
                ]
                for cp in cps:
                    cp.wait()

                def body(j, carry):
                    r0 = j * LANES
                    for k in range(LANES):
                        t = r0 + k
                        d = jnp.zeros((LANES,), jnp.float32)
                        for g in range(KE_DIM // LANES):
                            sl = pl.ds(g * LANES, LANES)
                            phm_v = phm[t, sl]
                            phv_v = phv[t, sl]
                            ptm_v = ptm[t, sl]
                            ptv_v = ptv[t, sl]
                            prm_v = prm[t, sl]
                            prv_v = prv[t, sl]
                            evp = ptv_v + phv_v
                            dp = prm_v - (ptm_v - phm_v)
                            ddp = dp * dp
                            sp = (evp + ddp) * (one / prv_v) + (prv_v + ddp) * (one / evp)

                            nhm_v = nhm[t, sl]
                            nhv_v = nhv[t, sl]
                            ntm_v = ntm[t, sl]
                            ntv_v = ntv[t, sl]
                            nrm_v = nrm[t, sl]
                            nrv_v = nrv[t, sl]
                            evn = ntv_v + nhv_v
                            dn = nrm_v - (ntm_v - nhm_v)
                            ddn = dn * dn
                            sn = (evn + ddn) * (one / nrv_v) + (nrv_v + ddn) * (one / evn)
                            d = d + (sp - sn)
                        p0 = d[0] + d[1]
                        p1 = d[2] + d[3]
                        p2 = d[4] + d[5]
                        p3 = d[6] + d[7]
                        p4 = d[8] + d[9]
                        p5 = d[10] + d[11]
                        p6 = d[12] + d[13]
                        p7 = d[14] + d[15]
                        tot = ((p0 + p1) + (p2 + p3)) + ((p4 + p5) + (p6 + p7))
                        carry = carry + jnp.maximum(
                            tot * jnp.float32(0.25) + jnp.float32(MARGIN_V),
                            jnp.float32(0.0))
                    return carry

                loss = lax.fori_loop(0, GR // LANES, body, loss)
            return loss

        loss = lax.fori_loop(0, NSTAGE, do_stage, jnp.float32(0.0))

        accv[...] = jnp.where(iota == 0, loss, jnp.float32(0.0))
        pltpu.sync_copy(accv, out.at[wid])

    return sc_fn


_SC_FN = _make_sc_call()


@jax.jit
def kernel(posX, negX, entityEmbed, entityCovar, relationEmbed, relationCovar):
    X = jnp.concatenate([posX, negX], axis=0)
    h = X[:, 0]
    r = X[:, 1]
    t = X[:, 2]
    partials = _SC_FN(entityEmbed, entityCovar, relationEmbed, relationCovar,
                      h, r, t)
    return jnp.sum(partials) / jnp.float32(posX.shape[0])


# speedup vs baseline: 1.0630x; 1.0103x over previous
import functools

import jax
import jax.numpy as jnp
from jax import lax
from jax.experimental import pallas as pl
from jax.experimental.pallas import tpu as pltpu
from jax.experimental.pallas import tpu_sc as plsc

KE_DIM = 64
MARGIN_V = 1.0
LANES = 16
NWORK = 32
GR = 64
POS_N = 16384
PER_W = POS_N // NWORK
NROUND = PER_W // GR


def _make_sc_call():
    mesh = plsc.VectorSubcoreMesh(core_axis_name="c", subcore_axis_name="s")

    row_t = pltpu.VMEM((GR, KE_DIM), jnp.float32)
    idx_t = pltpu.VMEM((PER_W,), jnp.int32)

    @functools.partial(
        pl.kernel,
        mesh=mesh,
        compiler_params=pltpu.CompilerParams(use_tc_tiling_on_sc=False),
        out_type=jax.ShapeDtypeStruct((NWORK, LANES), jnp.float32),
        scratch_types=[
            idx_t, idx_t, idx_t,
            idx_t, idx_t, idx_t,
            row_t, row_t, row_t, row_t, row_t, row_t,
            row_t, row_t, row_t, row_t, row_t, row_t,
            row_t, row_t, row_t, row_t, row_t, row_t,
            row_t, row_t, row_t, row_t, row_t, row_t,
            pltpu.VMEM((LANES,), jnp.float32),
            pltpu.SemaphoreType.DMA,
            pltpu.SemaphoreType.DMA,
        ],
    )
    def sc_fn(eEmb, eCov, rEmb, rCov, hIdx, rIdx, tIdx, out,
              phix, prix, ptix, nhix, nrix, ntix,
              a0, a1, a2, a3, a4, a5, a6, a7, a8, a9, a10, a11,
              b0, b1, b2, b3, b4, b5, b6, b7, b8, b9, b10, b11,
              accv, sem0, sem1):
        cid = lax.axis_index("c")
        sid = lax.axis_index("s")
        wid = sid * 2 + cid
        base = wid * PER_W

        iota = lax.iota(jnp.int32, LANES)
        one = jnp.float32(1.0)

        pltpu.sync_copy(hIdx.at[pl.ds(base, PER_W)], phix)
        pltpu.sync_copy(rIdx.at[pl.ds(base, PER_W)], prix)
        pltpu.sync_copy(tIdx.at[pl.ds(base, PER_W)], ptix)
        neg_base = base + POS_N
        pltpu.sync_copy(hIdx.at[pl.ds(neg_base, PER_W)], nhix)
        pltpu.sync_copy(rIdx.at[pl.ds(neg_base, PER_W)], nrix)
        pltpu.sync_copy(tIdx.at[pl.ds(neg_base, PER_W)], ntix)

        sets = [
            (a0, a1, a2, a3, a4, a5, a6, a7, a8, a9, a10, a11),
            (b0, b1, b2, b3, b4, b5, b6, b7, b8, b9, b10, b11),
        ]
        sems = [sem0, sem1]

        def issue(r, bufs, sem):
            sl_i = pl.ds(r * GR, GR)
            (phm, phv, ptm, ptv, prm, prv,
             nhm, nhv, ntm, ntv, nrm, nrv) = bufs
            return [
                pltpu.async_copy(eEmb.at[phix.at[sl_i]], phm, sem),
                pltpu.async_copy(eCov.at[phix.at[sl_i]], phv, sem),
                pltpu.async_copy(eEmb.at[ptix.at[sl_i]], ptm, sem),
                pltpu.async_copy(eCov.at[ptix.at[sl_i]], ptv, sem),
                pltpu.async_copy(rEmb.at[prix.at[sl_i]], prm, sem),
                pltpu.async_copy(rCov.at[prix.at[sl_i]], prv, sem),
                pltpu.async_copy(eEmb.at[nhix.at[sl_i]], nhm, sem),
                pltpu.async_copy(eCov.at[nhix.at[sl_i]], nhv, sem),
                pltpu.async_copy(eEmb.at[ntix.at[sl_i]], ntm, sem),
                pltpu.async_copy(eCov.at[ntix.at[sl_i]], ntv, sem),
                pltpu.async_copy(rEmb.at[nrix.at[sl_i]], nrm, sem),
                pltpu.async_copy(rCov.at[nrix.at[sl_i]], nrv, sem),
            ]

        def compute(bufs, loss):
            (phm, phv, ptm, ptv, prm, prv,
             nhm, nhv, ntm, ntv, nrm, nrv) = bufs

            def body(j, carry):
                r0 = j * LANES
                for k in range(LANES):
                    t = r0 + k
                    d = jnp.zeros((LANES,), jnp.float32)
                    for g in range(KE_DIM // LANES):
                        sl = pl.ds(g * LANES, LANES)
                        phm_v = phm[t, sl]
                        phv_v = phv[t, sl]
                        ptm_v = ptm[t, sl]
                        ptv_v = ptv[t, sl]
                        prm_v = prm[t, sl]
                        prv_v = prv[t, sl]
                        evp = ptv_v + phv_v
                        dp = prm_v - (ptm_v - phm_v)
                        ddp = dp * dp
                        sp = (evp + ddp) * (one / prv_v) + (prv_v + ddp) * (one / evp)

                        nhm_v = nhm[t, sl]
                        nhv_v = nhv[t, sl]
                        ntm_v = ntm[t, sl]
                        ntv_v = ntv[t, sl]
                        nrm_v = nrm[t, sl]
                        nrv_v = nrv[t, sl]
                        evn = ntv_v + nhv_v
                        dn = nrm_v - (ntm_v - nhm_v)
                        ddn = dn * dn
                        sn = (evn + ddn) * (one / nrv_v) + (nrv_v + ddn) * (one / evn)
                        d = d + (sp - sn)
                    p0 = d[0] + d[1]
                    p1 = d[2] + d[3]
                    p2 = d[4] + d[5]
                    p3 = d[6] + d[7]
                    p4 = d[8] + d[9]
                    p5 = d[10] + d[11]
                    p6 = d[12] + d[13]
                    p7 = d[14] + d[15]
                    tot = ((p0 + p1) + (p2 + p3)) + ((p4 + p5) + (p6 + p7))
                    carry = carry + jnp.maximum(
                        tot * jnp.float32(0.25) + jnp.float32(MARGIN_V),
                        jnp.float32(0.0))
                return carry

            return lax.fori_loop(0, GR // LANES, body, loss)

        def drain(bufs, sem):
            for buf in bufs:
                pltpu.make_async_copy(eEmb.at[pl.ds(0, GR)], buf, sem).wait()

        loss = jnp.float32(0.0)
        issue(0, sets[0], sems[0])
        issue(1, sets[1], sems[1])

        def pair_body(i, loss):
            r = 2 * i
            drain(sets[0], sems[0])
            loss = compute(sets[0], loss)
            issue(r + 2, sets[0], sems[0])
            drain(sets[1], sems[1])
            loss = compute(sets[1], loss)
            issue(r + 3, sets[1], sems[1])
            return loss

        loss = lax.fori_loop(0, NROUND // 2 - 1, pair_body, loss)
        drain(sets[0], sems[0])
        loss = compute(sets[0], loss)
        drain(sets[1], sems[1])
        loss = compute(sets[1], loss)

        accv[...] = jnp.where(iota == 0, loss, jnp.float32(0.0))
        pltpu.sync_copy(accv, out.at[wid])

    return sc_fn


_SC_FN = _make_sc_call()


@jax.jit
def kernel(posX, negX, entityEmbed, entityCovar, relationEmbed, relationCovar):
    X = jnp.concatenate([posX, negX], axis=0)
    h = X[:, 0]
    r = X[:, 1]
    t = X[:, 2]
    partials = _SC_FN(entityEmbed, entityCovar, relationEmbed, relationCovar,
                      h, r, t)
    return jnp.sum(partials) / jnp.float32(posX.shape[0])
